# baseline (speedup 1.0000x reference)
import functools

import jax
import jax.numpy as jnp
from jax import lax
from jax.experimental import pallas as pl
from jax.experimental.pallas import tpu as pltpu

K = 1024
DIM = 2048
KNN = 8
Q = 32
NOPS = 2
KT = 256
GROUP = K // NOPS
NSTEPS = GROUP // KT


def _tc_body(query_ref, a0_ref, a1_ref, label_ref, out_ref, dist_ref, logq_ref):
    i = pl.program_id(0)

    @pl.when(i == 0)
    def _():
        logq_ref[...] = jnp.log(query_ref[...])

    for g, a_ref in enumerate((a0_ref, a1_ref)):
        a = a_ref[...]
        log_a = jnp.log(a)
        self_term = jnp.sum(a * log_a, axis=1, keepdims=True)
        cross = lax.dot_general(
            logq_ref[...], a, (((1,), (1,)), ((), ())),
            preferred_element_type=jnp.float32)
        self_row = lax.transpose(self_term, (1, 0))
        dist_ref[:, pl.ds(g * GROUP + i * KT, KT)] = (self_row - cross) / DIM

    @pl.when(i == NSTEPS - 1)
    def _():
        d = dist_ref[...]
        iota = lax.broadcasted_iota(jnp.int32, (Q, K), 1)
        enc = iota * 2 + label_ref[...]
        s = jnp.zeros((Q, 1), jnp.int32)
        big = 2 * K + 2
        for _ in range(KNN):
            m = jnp.min(d, axis=1, keepdims=True)
            e = jnp.min(jnp.where(d == m, enc, big), axis=1, keepdims=True)
            s = s + (e & 1)
            d = jnp.where(enc == e, jnp.inf, d)
        out_ref[...] = lax.transpose(
            (s >= KNN // 2 + 1).astype(jnp.int32), (1, 0))


@jax.jit
def kernel(query, queue_anchor, queue_label):
    labels_2d = queue_label.reshape(1, K)
    out = pl.pallas_call(
        _tc_body,
        grid=(NSTEPS,),
        in_specs=[
            pl.BlockSpec((Q, DIM), lambda i: (0, 0)),
            pl.BlockSpec((KT, DIM), lambda i: (i, 0)),
            pl.BlockSpec((KT, DIM), lambda i: (i + NSTEPS, 0)),
            pl.BlockSpec((1, K), lambda i: (0, 0)),
        ],
        out_specs=pl.BlockSpec((1, Q), lambda i: (0, 0)),
        out_shape=jax.ShapeDtypeStruct((1, Q), jnp.int32),
        scratch_shapes=[
            pltpu.VMEM((Q, K), jnp.float32),
            pltpu.VMEM((Q, DIM), jnp.float32),
        ],
    )(query, queue_anchor, queue_anchor, labels_2d)
    return out.reshape(Q)


# speedup vs baseline: 1.0761x; 1.0761x over previous
import functools

import jax
import jax.numpy as jnp
from jax import lax
from jax.experimental import pallas as pl
from jax.experimental.pallas import tpu as pltpu

K = 1024
DIM = 2048
KNN = 8
Q = 32
KT = 512
NSTEPS = K // KT


def _tc_body(query_ref, anchor_ref, label_ref, out_ref, dist_ref, logq_ref):
    i = pl.program_id(0)

    @pl.when(i == 0)
    def _():
        logq_ref[...] = jnp.log(query_ref[...])

    a = anchor_ref[...]
    log_a = jnp.log(a)
    self_term = jnp.sum(a * log_a, axis=1, keepdims=True)
    cross = lax.dot_general(
        a, logq_ref[...], (((1,), (1,)), ((), ())),
        preferred_element_type=jnp.float32)
    dist_ref[:, pl.ds(i * KT, KT)] = lax.transpose(
        (self_term - cross) / DIM, (1, 0))

    @pl.when(i == NSTEPS - 1)
    def _():
        d = dist_ref[...]
        iota = lax.broadcasted_iota(jnp.int32, (Q, K), 1)
        enc = iota * 2 + label_ref[...]
        s = jnp.zeros((Q, 1), jnp.int32)
        big = 2 * K + 2
        for _ in range(KNN):
            m = jnp.min(d, axis=1, keepdims=True)
            e = jnp.min(jnp.where(d == m, enc, big), axis=1, keepdims=True)
            s = s + (e & 1)
            d = jnp.where(enc == e, jnp.inf, d)
        out_ref[...] = lax.transpose(
            (s >= KNN // 2 + 1).astype(jnp.int32), (1, 0))


@jax.jit
def kernel(query, queue_anchor, queue_label):
    labels_2d = queue_label.reshape(1, K)
    out = pl.pallas_call(
        _tc_body,
        grid=(NSTEPS,),
        in_specs=[
            pl.BlockSpec((Q, DIM), lambda i: (0, 0)),
            pl.BlockSpec((KT, DIM), lambda i: (i, 0)),
            pl.BlockSpec((1, K), lambda i: (0, 0)),
        ],
        out_specs=pl.BlockSpec((1, Q), lambda i: (0, 0)),
        out_shape=jax.ShapeDtypeStruct((1, Q), jnp.int32),
        scratch_shapes=[
            pltpu.VMEM((Q, K), jnp.float32),
            pltpu.VMEM((Q, DIM), jnp.float32),
        ],
    )(query, queue_anchor, labels_2d)
    return out.reshape(Q)
